# baseline (speedup 1.0000x reference)
import jax
import jax.numpy as jnp
from jax.experimental import pallas as pl
from jax.experimental.pallas import tpu as pltpu

B = 1024
T = 128
OBS = 256
A = 128

_BB = 32


def _store_kernel(step_ref,
                  obs_in, mask_in, act_in, rew_in, lp_in, val_in,
                  obs_new, mask_new, act_new, rew_new, lp_new, val_new,
                  obs_out, mask_out, act_out, rew_out, lp_out, val_out):
    s = step_ref[0]
    t3 = jax.lax.broadcasted_iota(jnp.int32, (1, T, 1), 1)
    hit3 = t3 == s
    obs_out[...] = jnp.where(hit3, obs_new[...][:, None, :], obs_in[...])
    m_in = mask_in[...].astype(jnp.int8)
    m_new = mask_new[...].astype(jnp.int8)[:, None, :]
    mask_out[...] = jnp.where(hit3, m_new, m_in) != 0
    t2 = jax.lax.broadcasted_iota(jnp.int32, (1, T), 1)
    hit2 = t2 == s
    act_out[...] = jnp.where(hit2, act_new[...], act_in[...])
    rew_out[...] = jnp.where(hit2, rew_new[...], rew_in[...])
    lp_out[...] = jnp.where(hit2, lp_new[...], lp_in[...])
    t2v = jax.lax.broadcasted_iota(jnp.int32, (1, T + 1), 1)
    val_out[...] = jnp.where(t2v == s, val_new[...], val_in[...])


def kernel(state_obs, state_action_mask, state_actions, state_rewards,
           state_log_prob, state_values, state_advantages, state_targets,
           step, obs, action_mask, action, reward, log_prob, value):
    step_arr = jnp.asarray(step, jnp.int32).reshape((1,))
    grid = (B // _BB,)

    def b3(t_, a_):
        return pl.BlockSpec((_BB, t_, a_), lambda i: (i, 0, 0))

    def b2(t_):
        return pl.BlockSpec((_BB, t_), lambda i: (i, 0))

    out_shapes = (
        jax.ShapeDtypeStruct((B, T, OBS), jnp.float32),
        jax.ShapeDtypeStruct((B, T, A), jnp.bool_),
        jax.ShapeDtypeStruct((B, T), jnp.int32),
        jax.ShapeDtypeStruct((B, T), jnp.float32),
        jax.ShapeDtypeStruct((B, T), jnp.float32),
        jax.ShapeDtypeStruct((B, T + 1), jnp.float32),
    )
    new_obs, new_mask, new_act, new_rew, new_lp, new_val = pl.pallas_call(
        _store_kernel,
        grid=grid,
        in_specs=[
            pl.BlockSpec(memory_space=pltpu.SMEM),
            b3(T, OBS), b3(T, A), b2(T), b2(T), b2(T), b2(T + 1),
            b2(OBS), b2(A), b2(1), b2(1), b2(1), b2(1),
        ],
        out_specs=[b3(T, OBS), b3(T, A), b2(T), b2(T), b2(T), b2(T + 1)],
        out_shape=out_shapes,
    )(step_arr,
      state_obs, state_action_mask, state_actions, state_rewards,
      state_log_prob, state_values,
      obs, action_mask,
      action.reshape(B, 1), reward.reshape(B, 1),
      log_prob.reshape(B, 1), value.reshape(B, 1))

    return (new_obs, new_mask, new_act, new_rew, new_lp, new_val,
            state_advantages, state_targets)


# speedup vs baseline: 1.0033x; 1.0033x over previous
import jax
import jax.numpy as jnp
from jax.experimental import pallas as pl
from jax.experimental.pallas import tpu as pltpu

B = 1024
T = 128
OBS = 256
A = 128

_BB = 64


def _store_kernel(step_ref,
                  obs_in, mask_in, act_in, rew_in, lp_in, val_in,
                  obs_new, mask_new, act_new, rew_new, lp_new, val_new,
                  obs_out, mask_out, act_out, rew_out, lp_out, val_out):
    s = step_ref[0]
    t3 = jax.lax.broadcasted_iota(jnp.int32, (1, T, 1), 1)
    hit3 = t3 == s
    obs_out[...] = jnp.where(hit3, obs_new[...][:, None, :], obs_in[...])
    m_in = mask_in[...].astype(jnp.int8)
    m_new = mask_new[...].astype(jnp.int8)[:, None, :]
    mask_out[...] = jnp.where(hit3, m_new, m_in) != 0
    t2 = jax.lax.broadcasted_iota(jnp.int32, (1, T), 1)
    hit2 = t2 == s
    act_out[...] = jnp.where(hit2, act_new[...], act_in[...])
    rew_out[...] = jnp.where(hit2, rew_new[...], rew_in[...])
    lp_out[...] = jnp.where(hit2, lp_new[...], lp_in[...])
    t2v = jax.lax.broadcasted_iota(jnp.int32, (1, T + 1), 1)
    val_out[...] = jnp.where(t2v == s, val_new[...], val_in[...])


def kernel(state_obs, state_action_mask, state_actions, state_rewards,
           state_log_prob, state_values, state_advantages, state_targets,
           step, obs, action_mask, action, reward, log_prob, value):
    step_arr = jnp.asarray(step, jnp.int32).reshape((1,))
    grid = (B // _BB,)

    def b3(t_, a_):
        return pl.BlockSpec((_BB, t_, a_), lambda i: (i, 0, 0))

    def b2(t_):
        return pl.BlockSpec((_BB, t_), lambda i: (i, 0))

    out_shapes = (
        jax.ShapeDtypeStruct((B, T, OBS), jnp.float32),
        jax.ShapeDtypeStruct((B, T, A), jnp.bool_),
        jax.ShapeDtypeStruct((B, T), jnp.int32),
        jax.ShapeDtypeStruct((B, T), jnp.float32),
        jax.ShapeDtypeStruct((B, T), jnp.float32),
        jax.ShapeDtypeStruct((B, T + 1), jnp.float32),
    )
    new_obs, new_mask, new_act, new_rew, new_lp, new_val = pl.pallas_call(
        _store_kernel,
        grid=grid,
        in_specs=[
            pl.BlockSpec(memory_space=pltpu.SMEM),
            b3(T, OBS), b3(T, A), b2(T), b2(T), b2(T), b2(T + 1),
            b2(OBS), b2(A), b2(1), b2(1), b2(1), b2(1),
        ],
        out_specs=[b3(T, OBS), b3(T, A), b2(T), b2(T), b2(T), b2(T + 1)],
        out_shape=out_shapes,
    )(step_arr,
      state_obs, state_action_mask, state_actions, state_rewards,
      state_log_prob, state_values,
      obs, action_mask,
      action.reshape(B, 1), reward.reshape(B, 1),
      log_prob.reshape(B, 1), value.reshape(B, 1))

    return (new_obs, new_mask, new_act, new_rew, new_lp, new_val,
            state_advantages, state_targets)
